# baseline (speedup 1.0000x reference)
import functools

import jax
import jax.numpy as jnp
from jax import lax
from jax.experimental import pallas as pl
from jax.experimental.pallas import tpu as pltpu
from jax.experimental.pallas import tpu_sc as plsc

N = 10000
E = 320000
D = 128
G = 16
NP = 10240
NW = 32
W = 128
NWIN = 84
EPW = NWIN * W
EP = NW * EPW
NBUF = 2
NIDX = 3
RPT = NP // 16
R = 1024
GRID = NP // R


def _seg_body(with_cnt, *refs):
    if with_cnt:
        (z_hbm, eidx, zrows_hbm, zcnt_hbm,
         s_out, cnt_out,
         acc_sh, cnt_sh, ones_v, cbuf) = refs[:10]
        rest = refs[10:]
    else:
        (z_hbm, eidx, zrows_hbm,
         s_out,
         acc_sh) = refs[:5]
        rest = refs[5:]
    rows = rest[:NBUF]
    idxb = rest[NBUF:NBUF + NIDX]
    semG = rest[NBUF + NIDX:2 * NBUF + NIDX]
    semS = rest[2 * NBUF + NIDX:3 * NBUF + NIDX]
    semI = rest[3 * NBUF + NIDX:3 * NBUF + 2 * NIDX]

    cid = lax.axis_index("c")
    sid = lax.axis_index("s")
    wid = sid * 2 + cid
    row0 = sid * RPT

    def waitG(b):
        pltpu.make_async_copy(z_hbm.at[pl.ds(0, W)], rows[b], semG[b]).wait()

    def waitS(b):
        pltpu.make_async_copy(z_hbm.at[pl.ds(0, W)], rows[b], semS[b]).wait()

    def loadI(w, i):
        pltpu.async_copy(eidx.at[wid, w], idxb[i], semI[i])

    def waitI(i):
        pltpu.make_async_copy(eidx.at[0, 0], idxb[i], semI[i]).wait()

    def issueG(w, b, i):
        pltpu.async_copy(z_hbm.at[idxb[i].at[0]], rows[b], semG[b])

    pltpu.sync_copy(zrows_hbm, rows[0])
    def zero_body(c, carry):
        pltpu.sync_copy(rows[0], acc_sh.at[pl.ds(row0 + c * W, W)])
        return carry
    lax.fori_loop(0, RPT // W, zero_body, 0)
    if with_cnt:
        pltpu.sync_copy(zcnt_hbm, cbuf)
        pltpu.sync_copy(cbuf, cnt_sh.at[pl.ds(row0, RPT)])
        for c in range(W // 16):
            ones_v[pl.ds(c * 16, 16)] = jnp.ones((16,), jnp.float32)

    pltpu.sync_copy(eidx.at[wid, 0], idxb[0])
    issueG(0, 0, 0)
    loadI(1, 1)
    plsc.subcore_barrier()

    def group_body(g, carry):
        for b6 in range(6):
            w = g * 6 + b6
            b = b6 % 2
            o = 1 - b
            i0 = b6 % 3
            i1 = (b6 + 1) % 3
            i2 = (b6 + 2) % 3
            waitG(b)
            pltpu.async_copy(rows[b], acc_sh.at[idxb[i0].at[1]], semS[b],
                             add=True)
            if with_cnt:
                pltpu.sync_copy(ones_v, cnt_sh.at[idxb[i0].at[1]], add=True)
            if b6 == 0:
                @pl.when(g > 0)
                def _():
                    waitS(o)
            else:
                waitS(o)
            if b6 < 4:
                loadI(w + 2, i2)
            else:
                @pl.when(g < NWIN // 6 - 1)
                def _():
                    loadI(w + 2, i2)
            if b6 < 5:
                waitI(i1)
                issueG(w + 1, o, i1)
            else:
                @pl.when(g < NWIN // 6 - 1)
                def _():
                    waitI(i1)
                    issueG(w + 1, o, i1)
        return carry
    lax.fori_loop(0, NWIN // 6, group_body, 0)
    waitS((NWIN - 1) % 2)
    plsc.subcore_barrier()

    def out_body(c, carry):
        r0 = row0 + c * W
        pltpu.sync_copy(acc_sh.at[pl.ds(r0, W)], rows[0])
        pltpu.sync_copy(rows[0], s_out.at[cid, pl.ds(r0, W)])
        return carry
    lax.fori_loop(0, RPT // W, out_body, 0)
    if with_cnt:
        pltpu.sync_copy(cnt_sh.at[pl.ds(row0, RPT)], cbuf)
        pltpu.sync_copy(cbuf, cnt_out.at[cid, pl.ds(row0, RPT)])


@functools.lru_cache(maxsize=None)
def _make_seg_kernel(with_cnt):
    mesh = plsc.VectorSubcoreMesh(core_axis_name="c", subcore_axis_name="s")
    out_type = [jax.ShapeDtypeStruct((2, NP, D), jnp.float32)]
    scratch = [
        pltpu.VMEM_SHARED((NP, D), jnp.float32),
    ]
    if with_cnt:
        out_type.append(jax.ShapeDtypeStruct((2, NP), jnp.float32))
        scratch.append(pltpu.VMEM_SHARED((NP,), jnp.float32))
        scratch += [
            pltpu.VMEM((W,), jnp.float32),
            pltpu.VMEM((RPT,), jnp.float32),
        ]
    scratch += [pltpu.VMEM((W, D), jnp.float32) for _ in range(NBUF)]
    scratch += [pltpu.VMEM((2, W), jnp.int32) for _ in range(NIDX)]
    scratch += [pltpu.SemaphoreType.DMA for _ in range(2 * NBUF + NIDX)]
    return pl.kernel(
        functools.partial(_seg_body, with_cnt),
        out_type=out_type,
        mesh=mesh,
        scratch_types=scratch,
    )


def _dotT(a, w):
    return lax.dot_general(a, w, (((1,), (1,)), ((), ())),
                           preferred_element_type=jnp.float32)


def _lin_first_body(x_ref, wl_ref, wr_ref, b_ref, z_ref, r_ref):
    x = x_ref[...]
    z_ref[...] = _dotT(x, wl_ref[...])
    r_ref[...] = _dotT(x, wr_ref[...]) + b_ref[...]


def _lin_mid_body(s_ref, cnt_ref, rp_ref, wl_ref, wr_ref, b_ref, z_ref, r_ref):
    inv = 1.0 / jnp.maximum(cnt_ref[0] + cnt_ref[1], 1.0)
    h = jnp.maximum((s_ref[0] + s_ref[1]) * inv + rp_ref[...], 0.0)
    z_ref[...] = _dotT(h, wl_ref[...])
    r_ref[...] = _dotT(h, wr_ref[...]) + b_ref[...]


def _final_body(s_ref, cnt_ref, rp_ref, batch_ref, wc_ref, bc_ref,
                out_ref, gs_acc, gc_acc):
    i = pl.program_id(0)
    inv = 1.0 / jnp.maximum(cnt_ref[0] + cnt_ref[1], 1.0)
    h = jnp.maximum((s_ref[0] + s_ref[1]) * inv + rp_ref[...], 0.0)
    brow = batch_ref[...].reshape(1, R)
    cls = lax.broadcasted_iota(jnp.int32, (G, 1), 0)
    oht = (brow == cls).astype(jnp.float32)

    @pl.when(i == 0)
    def _():
        gs_acc[...] = jnp.zeros_like(gs_acc)
        gc_acc[...] = jnp.zeros_like(gc_acc)

    gs_acc[...] += lax.dot_general(oht, h, (((1,), (0,)), ((), ())),
                                   preferred_element_type=jnp.float32)
    gc_acc[...] += jnp.sum(oht, axis=1, keepdims=True)

    @pl.when(i == GRID - 1)
    def _():
        pooled = gs_acc[...] / jnp.maximum(gc_acc[...], 1.0)
        prod = pooled * wc_ref[...]
        logit = jnp.sum(prod, axis=1, keepdims=True) + bc_ref[0, 0]
        out_ref[...] = jax.nn.sigmoid(logit)


def _full(shape):
    return pl.BlockSpec(shape, lambda i: (0,) * len(shape))


_row_spec = pl.BlockSpec((R, D), lambda i: (i, 0))
_s_spec = pl.BlockSpec((2, R, D), lambda i: (0, i, 0))
_cnt_spec = pl.BlockSpec((2, R, 1), lambda i: (0, i, 0))
_zr_out = [jax.ShapeDtypeStruct((NP, D), jnp.float32),
           jax.ShapeDtypeStruct((NP, D), jnp.float32)]

_lin_first = pl.pallas_call(
    _lin_first_body,
    grid=(GRID,),
    in_specs=[_row_spec, _full((D, D)), _full((D, D)), _full((1, D))],
    out_specs=[_row_spec, _row_spec],
    out_shape=_zr_out,
)

_lin_mid = pl.pallas_call(
    _lin_mid_body,
    grid=(GRID,),
    in_specs=[_s_spec, _cnt_spec, _row_spec,
              _full((D, D)), _full((D, D)), _full((1, D))],
    out_specs=[_row_spec, _row_spec],
    out_shape=_zr_out,
)

_final = pl.pallas_call(
    _final_body,
    grid=(GRID,),
    in_specs=[_s_spec, _cnt_spec, _row_spec,
              pl.BlockSpec((1, 1, R), lambda i: (i, 0, 0)),
              _full((1, D)), _full((1, 1))],
    out_specs=_full((G, 1)),
    out_shape=jax.ShapeDtypeStruct((G, 1), jnp.float32),
    scratch_shapes=[pltpu.VMEM((G, D), jnp.float32),
                    pltpu.VMEM((G, 1), jnp.float32)],
)

def kernel(x, edge_index, edge_weight, batch,
           W1l, b1l, W1r, W2l, b2l, W2r, W3l, b3l, W3r, W4l, b4l, W4r,
           Wc, bc):
    del edge_weight
    src = edge_index[0]
    dst = edge_index[1]
    npad = EP - E
    src_p = jnp.concatenate(
        [src, jnp.zeros((npad,), jnp.int32)]).reshape(NW, NWIN, W)
    dst_p = jnp.concatenate(
        [dst, N + (jnp.arange(npad, dtype=jnp.int32) % (NP - N))]
    ).reshape(NW, NWIN, W)
    eidx = jnp.stack([src_p, dst_p], axis=2)
    batch_p = jnp.concatenate(
        [batch, jnp.full((NP - N,), G, jnp.int32)]).reshape(GRID, 1, R)
    x_p = jnp.pad(x, ((0, NP - N), (0, 0)))
    b1 = b1l.reshape(1, D)
    b2 = b2l.reshape(1, D)
    b3 = b3l.reshape(1, D)
    b4 = b4l.reshape(1, D)
    bc2 = bc.reshape(1, 1)
    zrows = jnp.zeros((W, D), jnp.float32)
    zcnt = jnp.zeros((RPT,), jnp.float32)

    _seg_first = _make_seg_kernel(True)
    _seg_rest = _make_seg_kernel(False)

    z1, r1 = _lin_first(x_p, W1l, W1r, b1)
    s1, cnt = _seg_first(z1, eidx, zrows, zcnt)
    cnt = cnt.reshape(2, NP, 1)
    z2, r2 = _lin_mid(s1, cnt, r1, W2l, W2r, b2)
    s2 = _seg_rest(z2, eidx, zrows)[0]
    z3, r3 = _lin_mid(s2, cnt, r2, W3l, W3r, b3)
    s3 = _seg_rest(z3, eidx, zrows)[0]
    z4, r4 = _lin_mid(s3, cnt, r3, W4l, W4r, b4)
    s4 = _seg_rest(z4, eidx, zrows)[0]
    return _final(s4, cnt, r4, batch_p, Wc, bc2)


# speedup vs baseline: 1.9411x; 1.9411x over previous
import functools

import jax
import jax.numpy as jnp
from jax import lax
from jax.experimental import pallas as pl
from jax.experimental.pallas import tpu as pltpu
from jax.experimental.pallas import tpu_sc as plsc

N = 10000
E = 320000
D = 128
G = 16
NP = 10240
NW = 32
W = 128
NWIN = 80
EPW = NWIN * W
EP = NW * EPW
NBUF = 2
RPT = NP // 16
R = 1024
GRID = NP // R


def _seg_body(with_cnt, *refs):
    if with_cnt:
        (z_hbm, srcp, dstp, zrows_hbm, zcnt_hbm,
         s_out, cnt_out,
         acc_sh, cnt_sh, ones_v, cbuf) = refs[:11]
        rest = refs[11:]
    else:
        (z_hbm, srcp, dstp, zrows_hbm,
         s_out,
         acc_sh) = refs[:6]
        rest = refs[6:]
    rows = rest[:NBUF]
    sidx = rest[NBUF:2 * NBUF]
    didx = rest[2 * NBUF]
    semG = rest[2 * NBUF + 1:3 * NBUF + 1]

    cid = lax.axis_index("c")
    sid = lax.axis_index("s")
    wid = sid * 2 + cid
    row0 = sid * RPT
    ebase = wid * EPW

    def waitG(b):
        pltpu.make_async_copy(z_hbm.at[pl.ds(0, W)], rows[b], semG[b]).wait()

    pltpu.sync_copy(zrows_hbm, rows[0])
    def zero_body(c, carry):
        pltpu.sync_copy(rows[0], acc_sh.at[pl.ds(row0 + c * W, W)])
        return carry
    lax.fori_loop(0, RPT // W, zero_body, 0)
    if with_cnt:
        pltpu.sync_copy(zcnt_hbm, cbuf)
        pltpu.sync_copy(cbuf, cnt_sh.at[pl.ds(row0, RPT)])
        for c in range(W // 16):
            ones_v[pl.ds(c * 16, 16)] = jnp.ones((16,), jnp.float32)

    pltpu.sync_copy(srcp.at[pl.ds(ebase, W)], sidx[0])
    pltpu.async_copy(z_hbm.at[sidx[0]], rows[0], semG[0])
    plsc.subcore_barrier()

    def group_body(g, carry):
        for b in range(2):
            w = g * 2 + b
            o = 1 - b
            pltpu.sync_copy(srcp.at[pl.ds(ebase + (w + 1) * W, W)], sidx[o])
            waitG(b)
            pltpu.async_copy(z_hbm.at[sidx[o]], rows[o], semG[o])
            pltpu.sync_copy(dstp.at[pl.ds(ebase + w * W, W)], didx)
            pltpu.sync_copy(rows[b], acc_sh.at[didx], add=True)
            if with_cnt:
                pltpu.sync_copy(ones_v, cnt_sh.at[didx], add=True)
        return carry
    lax.fori_loop(0, NWIN // 2, group_body, 0)
    waitG(NWIN % 2)
    plsc.subcore_barrier()

    def out_body(c, carry):
        r0 = row0 + c * W
        pltpu.sync_copy(acc_sh.at[pl.ds(r0, W)], rows[0])
        pltpu.sync_copy(rows[0], s_out.at[cid, pl.ds(r0, W)])
        return carry
    lax.fori_loop(0, RPT // W, out_body, 0)
    if with_cnt:
        pltpu.sync_copy(cnt_sh.at[pl.ds(row0, RPT)], cbuf)
        pltpu.sync_copy(cbuf, cnt_out.at[cid, pl.ds(row0, RPT)])


@functools.lru_cache(maxsize=None)
def _make_seg_kernel(with_cnt):
    mesh = plsc.VectorSubcoreMesh(core_axis_name="c", subcore_axis_name="s")
    out_type = [jax.ShapeDtypeStruct((2, NP, D), jnp.float32)]
    scratch = [
        pltpu.VMEM_SHARED((NP, D), jnp.float32),
    ]
    if with_cnt:
        out_type.append(jax.ShapeDtypeStruct((2, NP), jnp.float32))
        scratch.append(pltpu.VMEM_SHARED((NP,), jnp.float32))
        scratch += [
            pltpu.VMEM((W,), jnp.float32),
            pltpu.VMEM((RPT,), jnp.float32),
        ]
    scratch += [pltpu.VMEM((W, D), jnp.float32) for _ in range(NBUF)]
    scratch += [pltpu.VMEM((W,), jnp.int32) for _ in range(NBUF)]
    scratch.append(pltpu.VMEM((W,), jnp.int32))
    scratch += [pltpu.SemaphoreType.DMA for _ in range(NBUF)]
    return pl.kernel(
        functools.partial(_seg_body, with_cnt),
        out_type=out_type,
        mesh=mesh,
        scratch_types=scratch,
    )


def _dotT(a, w):
    return lax.dot_general(a, w, (((1,), (1,)), ((), ())),
                           preferred_element_type=jnp.float32)


def _lin_first_body(x_ref, wl_ref, wr_ref, b_ref, z_ref, r_ref):
    x = x_ref[...]
    z_ref[...] = _dotT(x, wl_ref[...])
    r_ref[...] = _dotT(x, wr_ref[...]) + b_ref[...]


def _lin_mid_body(s_ref, cnt_ref, rp_ref, wl_ref, wr_ref, b_ref, z_ref, r_ref):
    inv = 1.0 / jnp.maximum(cnt_ref[0] + cnt_ref[1], 1.0)
    h = jnp.maximum((s_ref[0] + s_ref[1]) * inv + rp_ref[...], 0.0)
    z_ref[...] = _dotT(h, wl_ref[...])
    r_ref[...] = _dotT(h, wr_ref[...]) + b_ref[...]


def _final_body(s_ref, cnt_ref, rp_ref, batch_ref, wc_ref, bc_ref,
                out_ref, gs_acc, gc_acc):
    i = pl.program_id(0)
    inv = 1.0 / jnp.maximum(cnt_ref[0] + cnt_ref[1], 1.0)
    h = jnp.maximum((s_ref[0] + s_ref[1]) * inv + rp_ref[...], 0.0)
    brow = batch_ref[...].reshape(1, R)
    cls = lax.broadcasted_iota(jnp.int32, (G, 1), 0)
    oht = (brow == cls).astype(jnp.float32)

    @pl.when(i == 0)
    def _():
        gs_acc[...] = jnp.zeros_like(gs_acc)
        gc_acc[...] = jnp.zeros_like(gc_acc)

    gs_acc[...] += lax.dot_general(oht, h, (((1,), (0,)), ((), ())),
                                   preferred_element_type=jnp.float32)
    gc_acc[...] += jnp.sum(oht, axis=1, keepdims=True)

    @pl.when(i == GRID - 1)
    def _():
        pooled = gs_acc[...] / jnp.maximum(gc_acc[...], 1.0)
        prod = pooled * wc_ref[...]
        logit = jnp.sum(prod, axis=1, keepdims=True) + bc_ref[0, 0]
        out_ref[...] = jax.nn.sigmoid(logit)


def _full(shape):
    return pl.BlockSpec(shape, lambda i: (0,) * len(shape))


_row_spec = pl.BlockSpec((R, D), lambda i: (i, 0))
_s_spec = pl.BlockSpec((2, R, D), lambda i: (0, i, 0))
_cnt_spec = pl.BlockSpec((2, R, 1), lambda i: (0, i, 0))
_zr_out = [jax.ShapeDtypeStruct((NP, D), jnp.float32),
           jax.ShapeDtypeStruct((NP, D), jnp.float32)]

_lin_first = pl.pallas_call(
    _lin_first_body,
    grid=(GRID,),
    in_specs=[_row_spec, _full((D, D)), _full((D, D)), _full((1, D))],
    out_specs=[_row_spec, _row_spec],
    out_shape=_zr_out,
)

_lin_mid = pl.pallas_call(
    _lin_mid_body,
    grid=(GRID,),
    in_specs=[_s_spec, _cnt_spec, _row_spec,
              _full((D, D)), _full((D, D)), _full((1, D))],
    out_specs=[_row_spec, _row_spec],
    out_shape=_zr_out,
)

_final = pl.pallas_call(
    _final_body,
    grid=(GRID,),
    in_specs=[_s_spec, _cnt_spec, _row_spec,
              pl.BlockSpec((1, 1, R), lambda i: (i, 0, 0)),
              _full((1, D)), _full((1, 1))],
    out_specs=_full((G, 1)),
    out_shape=jax.ShapeDtypeStruct((G, 1), jnp.float32),
    scratch_shapes=[pltpu.VMEM((G, D), jnp.float32),
                    pltpu.VMEM((G, 1), jnp.float32)],
)

def kernel(x, edge_index, edge_weight, batch,
           W1l, b1l, W1r, W2l, b2l, W2r, W3l, b3l, W3r, W4l, b4l, W4r,
           Wc, bc):
    del edge_weight
    src = edge_index[0]
    dst = edge_index[1]
    npad = EP - E
    src_p = jnp.concatenate(
        [src, jnp.zeros((npad + W,), jnp.int32)])
    dst_p = jnp.concatenate(
        [dst, N + (jnp.arange(npad, dtype=jnp.int32) % (NP - N))])
    batch_p = jnp.concatenate(
        [batch, jnp.full((NP - N,), G, jnp.int32)]).reshape(GRID, 1, R)
    x_p = jnp.pad(x, ((0, NP - N), (0, 0)))
    b1 = b1l.reshape(1, D)
    b2 = b2l.reshape(1, D)
    b3 = b3l.reshape(1, D)
    b4 = b4l.reshape(1, D)
    bc2 = bc.reshape(1, 1)
    zrows = jnp.zeros((W, D), jnp.float32)
    zcnt = jnp.zeros((RPT,), jnp.float32)

    _seg_first = _make_seg_kernel(True)
    _seg_rest = _make_seg_kernel(False)

    z1, r1 = _lin_first(x_p, W1l, W1r, b1)
    s1, cnt = _seg_first(z1, src_p, dst_p, zrows, zcnt)
    cnt = cnt.reshape(2, NP, 1)
    z2, r2 = _lin_mid(s1, cnt, r1, W2l, W2r, b2)
    s2 = _seg_rest(z2, src_p, dst_p, zrows)[0]
    z3, r3 = _lin_mid(s2, cnt, r2, W3l, W3r, b3)
    s3 = _seg_rest(z3, src_p, dst_p, zrows)[0]
    z4, r4 = _lin_mid(s3, cnt, r3, W4l, W4r, b4)
    s4 = _seg_rest(z4, src_p, dst_p, zrows)[0]
    return _final(s4, cnt, r4, batch_p, Wc, bc2)


# speedup vs baseline: 4.2828x; 2.2064x over previous
import functools

import jax
import jax.numpy as jnp
from jax import lax
from jax.experimental import pallas as pl
from jax.experimental.pallas import tpu as pltpu
from jax.experimental.pallas import tpu_sc as plsc

N = 10000
E = 320000
D = 128
G = 16
NP = 10240
NW = 32
W = 128
NWIN = 80
EPW = NWIN * W
EP = NW * EPW
RPT = NP // 16
R = 1024
GRID = NP // R


def _seg_body(with_cnt, *refs):
    if with_cnt:
        (z_hbm, eidx, zrows_hbm, zcnt_hbm,
         s_out, cnt_out,
         acc_sh, cnt_sh, ones_v, cbuf, rows, idxb, sem) = refs
    else:
        (z_hbm, eidx, zrows_hbm,
         s_out,
         acc_sh, rows, idxb, sem) = refs

    cid = lax.axis_index("c")
    sid = lax.axis_index("s")
    wid = sid * 2 + cid
    row0 = sid * RPT

    pltpu.sync_copy(zrows_hbm, rows)
    def zero_body(c, carry):
        pltpu.sync_copy(rows, acc_sh.at[pl.ds(row0 + c * W, W)])
        return carry
    lax.fori_loop(0, RPT // W, zero_body, 0)
    if with_cnt:
        pltpu.sync_copy(zcnt_hbm, cbuf)
        pltpu.sync_copy(cbuf, cnt_sh.at[pl.ds(row0, RPT)])
        for c in range(W // 16):
            ones_v[pl.ds(c * 16, 16)] = jnp.ones((16,), jnp.float32)
    plsc.subcore_barrier()

    def win_body(w, carry):
        pltpu.sync_copy(eidx.at[wid, w], idxb)
        pltpu.async_copy(z_hbm.at[idxb.at[0]], rows, sem).wait()
        pltpu.sync_copy(rows, acc_sh.at[idxb.at[1]], add=True)
        if with_cnt:
            pltpu.sync_copy(ones_v, cnt_sh.at[idxb.at[1]], add=True)
        return carry
    lax.fori_loop(0, NWIN, win_body, 0)
    plsc.subcore_barrier()

    def out_body(c, carry):
        r0 = row0 + c * W
        pltpu.sync_copy(acc_sh.at[pl.ds(r0, W)], rows)
        pltpu.sync_copy(rows, s_out.at[cid, pl.ds(r0, W)])
        return carry
    lax.fori_loop(0, RPT // W, out_body, 0)
    if with_cnt:
        pltpu.sync_copy(cnt_sh.at[pl.ds(row0, RPT)], cbuf)
        pltpu.sync_copy(cbuf, cnt_out.at[cid, pl.ds(row0, RPT)])


@functools.lru_cache(maxsize=None)
def _make_seg_kernel(with_cnt):
    mesh = plsc.VectorSubcoreMesh(core_axis_name="c", subcore_axis_name="s")
    out_type = [jax.ShapeDtypeStruct((2, NP, D), jnp.float32)]
    scratch = [
        pltpu.VMEM_SHARED((NP, D), jnp.float32),
    ]
    if with_cnt:
        out_type.append(jax.ShapeDtypeStruct((2, NP), jnp.float32))
        scratch.append(pltpu.VMEM_SHARED((NP,), jnp.float32))
        scratch += [
            pltpu.VMEM((W,), jnp.float32),
            pltpu.VMEM((RPT,), jnp.float32),
        ]
    scratch += [
        pltpu.VMEM((W, D), jnp.float32),
        pltpu.VMEM((2, W), jnp.int32),
        pltpu.SemaphoreType.DMA,
    ]
    return pl.kernel(
        functools.partial(_seg_body, with_cnt),
        out_type=out_type,
        mesh=mesh,
        scratch_types=scratch,
    )


def _dotT(a, w):
    return lax.dot_general(a, w, (((1,), (1,)), ((), ())),
                           preferred_element_type=jnp.float32)


def _lin_first_body(x_ref, wl_ref, wr_ref, b_ref, z_ref, r_ref):
    x = x_ref[...]
    z_ref[...] = _dotT(x, wl_ref[...])
    r_ref[...] = _dotT(x, wr_ref[...]) + b_ref[...]


def _lin_mid_body(s_ref, cnt_ref, rp_ref, wl_ref, wr_ref, b_ref, z_ref, r_ref):
    inv = 1.0 / jnp.maximum(cnt_ref[0] + cnt_ref[1], 1.0)
    h = jnp.maximum((s_ref[0] + s_ref[1]) * inv + rp_ref[...], 0.0)
    z_ref[...] = _dotT(h, wl_ref[...])
    r_ref[...] = _dotT(h, wr_ref[...]) + b_ref[...]


def _final_body(s_ref, cnt_ref, rp_ref, batch_ref, wc_ref, bc_ref,
                out_ref, gs_acc, gc_acc):
    i = pl.program_id(0)
    inv = 1.0 / jnp.maximum(cnt_ref[0] + cnt_ref[1], 1.0)
    h = jnp.maximum((s_ref[0] + s_ref[1]) * inv + rp_ref[...], 0.0)
    brow = batch_ref[...].reshape(1, R)
    cls = lax.broadcasted_iota(jnp.int32, (G, 1), 0)
    oht = (brow == cls).astype(jnp.float32)

    @pl.when(i == 0)
    def _():
        gs_acc[...] = jnp.zeros_like(gs_acc)
        gc_acc[...] = jnp.zeros_like(gc_acc)

    gs_acc[...] += lax.dot_general(oht, h, (((1,), (0,)), ((), ())),
                                   preferred_element_type=jnp.float32)
    gc_acc[...] += jnp.sum(oht, axis=1, keepdims=True)

    @pl.when(i == GRID - 1)
    def _():
        pooled = gs_acc[...] / jnp.maximum(gc_acc[...], 1.0)
        prod = pooled * wc_ref[...]
        logit = jnp.sum(prod, axis=1, keepdims=True) + bc_ref[0, 0]
        out_ref[...] = jax.nn.sigmoid(logit)


def _full(shape):
    return pl.BlockSpec(shape, lambda i: (0,) * len(shape))


_row_spec = pl.BlockSpec((R, D), lambda i: (i, 0))
_s_spec = pl.BlockSpec((2, R, D), lambda i: (0, i, 0))
_cnt_spec = pl.BlockSpec((2, R, 1), lambda i: (0, i, 0))
_zr_out = [jax.ShapeDtypeStruct((NP, D), jnp.float32),
           jax.ShapeDtypeStruct((NP, D), jnp.float32)]

_lin_first = pl.pallas_call(
    _lin_first_body,
    grid=(GRID,),
    in_specs=[_row_spec, _full((D, D)), _full((D, D)), _full((1, D))],
    out_specs=[_row_spec, _row_spec],
    out_shape=_zr_out,
)

_lin_mid = pl.pallas_call(
    _lin_mid_body,
    grid=(GRID,),
    in_specs=[_s_spec, _cnt_spec, _row_spec,
              _full((D, D)), _full((D, D)), _full((1, D))],
    out_specs=[_row_spec, _row_spec],
    out_shape=_zr_out,
)

_final = pl.pallas_call(
    _final_body,
    grid=(GRID,),
    in_specs=[_s_spec, _cnt_spec, _row_spec,
              pl.BlockSpec((1, 1, R), lambda i: (i, 0, 0)),
              _full((1, D)), _full((1, 1))],
    out_specs=_full((G, 1)),
    out_shape=jax.ShapeDtypeStruct((G, 1), jnp.float32),
    scratch_shapes=[pltpu.VMEM((G, D), jnp.float32),
                    pltpu.VMEM((G, 1), jnp.float32)],
)

def kernel(x, edge_index, edge_weight, batch,
           W1l, b1l, W1r, W2l, b2l, W2r, W3l, b3l, W3r, W4l, b4l, W4r,
           Wc, bc):
    del edge_weight
    src = edge_index[0]
    dst = edge_index[1]
    npad = EP - E
    src_p = jnp.concatenate(
        [src, jnp.arange(npad, dtype=jnp.int32) % N]).reshape(NW, NWIN, W)
    dst_p = jnp.concatenate(
        [dst, N + (jnp.arange(npad, dtype=jnp.int32) % (NP - N))]
    ).reshape(NW, NWIN, W)
    eidx = jnp.stack([src_p, dst_p], axis=2)
    batch_p = jnp.concatenate(
        [batch, jnp.full((NP - N,), G, jnp.int32)]).reshape(GRID, 1, R)
    x_p = jnp.pad(x, ((0, NP - N), (0, 0)))
    b1 = b1l.reshape(1, D)
    b2 = b2l.reshape(1, D)
    b3 = b3l.reshape(1, D)
    b4 = b4l.reshape(1, D)
    bc2 = bc.reshape(1, 1)
    zrows = jnp.zeros((W, D), jnp.float32)
    zcnt = jnp.zeros((RPT,), jnp.float32)

    _seg_first = _make_seg_kernel(True)
    _seg_rest = _make_seg_kernel(False)

    z1, r1 = _lin_first(x_p, W1l, W1r, b1)
    s1, cnt = _seg_first(z1, eidx, zrows, zcnt)
    cnt = cnt.reshape(2, NP, 1)
    z2, r2 = _lin_mid(s1, cnt, r1, W2l, W2r, b2)
    s2 = _seg_rest(z2, eidx, zrows)[0]
    z3, r3 = _lin_mid(s2, cnt, r2, W3l, W3r, b3)
    s3 = _seg_rest(z3, eidx, zrows)[0]
    z4, r4 = _lin_mid(s3, cnt, r3, W4l, W4r, b4)
    s4 = _seg_rest(z4, eidx, zrows)[0]
    return _final(s4, cnt, r4, batch_p, Wc, bc2)


# speedup vs baseline: 6.5463x; 1.5285x over previous
import functools

import jax
import jax.numpy as jnp
from jax import lax
from jax.experimental import pallas as pl
from jax.experimental.pallas import tpu as pltpu
from jax.experimental.pallas import tpu_sc as plsc

N = 10000
E = 320000
D = 128
G = 16
NP = 10240
NW = 32
W = 128
NWIN = 80
EPW = NWIN * W
EP = NW * EPW
RPT = NP // 16
R = 1024
GRID = NP // R


def _seg_body(with_cnt, *refs):
    if with_cnt:
        (z_hbm, eidx, zrows_hbm, zcnt_hbm,
         s_out, cnt_out,
         acc_sh, cnt_sh, ones_v, cbuf,
         rows0, rows1, idxb0, idxb1, sem0, sem1) = refs
    else:
        (z_hbm, eidx, zrows_hbm,
         s_out,
         acc_sh, rows0, rows1, idxb0, idxb1, sem0, sem1) = refs
    rows = (rows0, rows1)
    idxb = (idxb0, idxb1)
    sem = (sem0, sem1)

    cid = lax.axis_index("c")
    sid = lax.axis_index("s")
    wid = sid * 2 + cid
    row0 = sid * RPT

    pltpu.sync_copy(zrows_hbm, rows[0])
    def zero_body(c, carry):
        pltpu.sync_copy(rows[0], acc_sh.at[pl.ds(row0 + c * W, W)])
        return carry
    lax.fori_loop(0, RPT // W, zero_body, 0)
    if with_cnt:
        pltpu.sync_copy(zcnt_hbm, cbuf)
        pltpu.sync_copy(cbuf, cnt_sh.at[pl.ds(row0, RPT)])
        for c in range(W // 16):
            ones_v[pl.ds(c * 16, 16)] = jnp.ones((16,), jnp.float32)
    pltpu.sync_copy(eidx.at[wid, 0], idxb[0])
    pltpu.async_copy(z_hbm.at[idxb[0].at[0]], rows[0], sem[0])
    plsc.subcore_barrier()

    def group_body(g, carry):
        for b in range(2):
            w = g * 2 + b
            o = 1 - b
            pltpu.sync_copy(eidx.at[wid, w + 1], idxb[o])
            pltpu.make_async_copy(
                z_hbm.at[pl.ds(0, W)], rows[b], sem[b]).wait()
            pltpu.async_copy(z_hbm.at[idxb[o].at[0]], rows[o], sem[o])
            pltpu.sync_copy(rows[b], acc_sh.at[idxb[b].at[1]], add=True)
            if with_cnt:
                pltpu.sync_copy(ones_v, cnt_sh.at[idxb[b].at[1]], add=True)
        return carry
    lax.fori_loop(0, NWIN // 2, group_body, 0)
    pltpu.make_async_copy(
        z_hbm.at[pl.ds(0, W)], rows[NWIN % 2], sem[NWIN % 2]).wait()
    plsc.subcore_barrier()

    def out_body(c, carry):
        r0 = row0 + c * W
        pltpu.sync_copy(acc_sh.at[pl.ds(r0, W)], rows[0])
        pltpu.sync_copy(rows[0], s_out.at[cid, pl.ds(r0, W)])
        return carry
    lax.fori_loop(0, RPT // W, out_body, 0)
    if with_cnt:
        pltpu.sync_copy(cnt_sh.at[pl.ds(row0, RPT)], cbuf)
        pltpu.sync_copy(cbuf, cnt_out.at[cid, pl.ds(row0, RPT)])


@functools.lru_cache(maxsize=None)
def _make_seg_kernel(with_cnt):
    mesh = plsc.VectorSubcoreMesh(core_axis_name="c", subcore_axis_name="s")
    out_type = [jax.ShapeDtypeStruct((2, NP, D), jnp.float32)]
    scratch = [
        pltpu.VMEM_SHARED((NP, D), jnp.float32),
    ]
    if with_cnt:
        out_type.append(jax.ShapeDtypeStruct((2, NP), jnp.float32))
        scratch.append(pltpu.VMEM_SHARED((NP,), jnp.float32))
        scratch += [
            pltpu.VMEM((W,), jnp.float32),
            pltpu.VMEM((RPT,), jnp.float32),
        ]
    scratch += [
        pltpu.VMEM((W, D), jnp.float32),
        pltpu.VMEM((W, D), jnp.float32),
        pltpu.VMEM((2, W), jnp.int32),
        pltpu.VMEM((2, W), jnp.int32),
        pltpu.SemaphoreType.DMA,
        pltpu.SemaphoreType.DMA,
    ]
    return pl.kernel(
        functools.partial(_seg_body, with_cnt),
        out_type=out_type,
        mesh=mesh,
        scratch_types=scratch,
    )


def _dotT(a, w):
    return lax.dot_general(a, w, (((1,), (1,)), ((), ())),
                           preferred_element_type=jnp.float32)


def _lin_first_body(x_ref, wl_ref, wr_ref, b_ref, z_ref, r_ref):
    x = x_ref[...]
    z_ref[...] = _dotT(x, wl_ref[...])
    r_ref[...] = _dotT(x, wr_ref[...]) + b_ref[...]


def _lin_mid_body(s_ref, cnt_ref, rp_ref, wl_ref, wr_ref, b_ref, z_ref, r_ref):
    inv = 1.0 / jnp.maximum(cnt_ref[0] + cnt_ref[1], 1.0)
    h = jnp.maximum((s_ref[0] + s_ref[1]) * inv + rp_ref[...], 0.0)
    z_ref[...] = _dotT(h, wl_ref[...])
    r_ref[...] = _dotT(h, wr_ref[...]) + b_ref[...]


def _final_body(s_ref, cnt_ref, rp_ref, batch_ref, wc_ref, bc_ref,
                out_ref, gs_acc, gc_acc):
    i = pl.program_id(0)
    inv = 1.0 / jnp.maximum(cnt_ref[0] + cnt_ref[1], 1.0)
    h = jnp.maximum((s_ref[0] + s_ref[1]) * inv + rp_ref[...], 0.0)
    brow = batch_ref[...].reshape(1, R)
    cls = lax.broadcasted_iota(jnp.int32, (G, 1), 0)
    oht = (brow == cls).astype(jnp.float32)

    @pl.when(i == 0)
    def _():
        gs_acc[...] = jnp.zeros_like(gs_acc)
        gc_acc[...] = jnp.zeros_like(gc_acc)

    gs_acc[...] += lax.dot_general(oht, h, (((1,), (0,)), ((), ())),
                                   preferred_element_type=jnp.float32)
    gc_acc[...] += jnp.sum(oht, axis=1, keepdims=True)

    @pl.when(i == GRID - 1)
    def _():
        pooled = gs_acc[...] / jnp.maximum(gc_acc[...], 1.0)
        prod = pooled * wc_ref[...]
        logit = jnp.sum(prod, axis=1, keepdims=True) + bc_ref[0, 0]
        out_ref[...] = jax.nn.sigmoid(logit)


def _full(shape):
    return pl.BlockSpec(shape, lambda i: (0,) * len(shape))


_row_spec = pl.BlockSpec((R, D), lambda i: (i, 0))
_s_spec = pl.BlockSpec((2, R, D), lambda i: (0, i, 0))
_cnt_spec = pl.BlockSpec((2, R, 1), lambda i: (0, i, 0))
_zr_out = [jax.ShapeDtypeStruct((NP, D), jnp.float32),
           jax.ShapeDtypeStruct((NP, D), jnp.float32)]

_lin_first = pl.pallas_call(
    _lin_first_body,
    grid=(GRID,),
    in_specs=[_row_spec, _full((D, D)), _full((D, D)), _full((1, D))],
    out_specs=[_row_spec, _row_spec],
    out_shape=_zr_out,
)

_lin_mid = pl.pallas_call(
    _lin_mid_body,
    grid=(GRID,),
    in_specs=[_s_spec, _cnt_spec, _row_spec,
              _full((D, D)), _full((D, D)), _full((1, D))],
    out_specs=[_row_spec, _row_spec],
    out_shape=_zr_out,
)

_final = pl.pallas_call(
    _final_body,
    grid=(GRID,),
    in_specs=[_s_spec, _cnt_spec, _row_spec,
              pl.BlockSpec((1, 1, R), lambda i: (i, 0, 0)),
              _full((1, D)), _full((1, 1))],
    out_specs=_full((G, 1)),
    out_shape=jax.ShapeDtypeStruct((G, 1), jnp.float32),
    scratch_shapes=[pltpu.VMEM((G, D), jnp.float32),
                    pltpu.VMEM((G, 1), jnp.float32)],
)

def kernel(x, edge_index, edge_weight, batch,
           W1l, b1l, W1r, W2l, b2l, W2r, W3l, b3l, W3r, W4l, b4l, W4r,
           Wc, bc):
    del edge_weight
    src = edge_index[0]
    dst = edge_index[1]
    npad = EP - E
    src_p = jnp.concatenate(
        [src, jnp.arange(npad, dtype=jnp.int32) % N]).reshape(NW, NWIN, W)
    dst_p = jnp.concatenate(
        [dst, N + (jnp.arange(npad, dtype=jnp.int32) % (NP - N))]
    ).reshape(NW, NWIN, W)
    eidx = jnp.stack([src_p, dst_p], axis=2)
    overrun = jnp.broadcast_to(
        (jnp.arange(2 * W, dtype=jnp.int32) % N).reshape(1, 1, 2, W),
        (NW, 1, 2, W))
    eidx = jnp.concatenate([eidx, overrun], axis=1)
    batch_p = jnp.concatenate(
        [batch, jnp.full((NP - N,), G, jnp.int32)]).reshape(GRID, 1, R)
    x_p = jnp.pad(x, ((0, NP - N), (0, 0)))
    b1 = b1l.reshape(1, D)
    b2 = b2l.reshape(1, D)
    b3 = b3l.reshape(1, D)
    b4 = b4l.reshape(1, D)
    bc2 = bc.reshape(1, 1)
    zrows = jnp.zeros((W, D), jnp.float32)
    zcnt = jnp.zeros((RPT,), jnp.float32)

    _seg_first = _make_seg_kernel(True)
    _seg_rest = _make_seg_kernel(False)

    z1, r1 = _lin_first(x_p, W1l, W1r, b1)
    s1, cnt = _seg_first(z1, eidx, zrows, zcnt)
    cnt = cnt.reshape(2, NP, 1)
    z2, r2 = _lin_mid(s1, cnt, r1, W2l, W2r, b2)
    s2 = _seg_rest(z2, eidx, zrows)[0]
    z3, r3 = _lin_mid(s2, cnt, r2, W3l, W3r, b3)
    s3 = _seg_rest(z3, eidx, zrows)[0]
    z4, r4 = _lin_mid(s3, cnt, r3, W4l, W4r, b4)
    s4 = _seg_rest(z4, eidx, zrows)[0]
    return _final(s4, cnt, r4, batch_p, Wc, bc2)
